# baseline (speedup 1.0000x reference)
import functools

import numpy as np

import jax
import jax.numpy as jnp
from jax import lax
from jax.experimental import pallas as pl
from jax.experimental.pallas import tpu as pltpu
from jax.experimental.pallas import tpu_sc as plsc

B = 2
C = 96
CP = C // 2
H = W = 224
HW = H * W
K = 9
P = 1792
NV = P // 16
NBLK = HW // P
NTASK = B * NBLK
CHUNK = 12544
WP = 256


def _round_bf16_bits(x):
    u = lax.bitcast_convert_type(x, jnp.uint32)
    return (u + jnp.uint32(0x8000)) >> 16


def _pack_body(x_ref, o_ref):
    x0 = x_ref[0, 0, 0]
    x1 = x_ref[0, 0, 1]
    word = (_round_bf16_bits(x0) << 16) | _round_bf16_bits(x1)
    wp = jnp.concatenate([word, jnp.zeros((56, WP - W), jnp.uint32)], axis=1)
    o_ref[0, 0, :] = lax.bitcast_convert_type(wp, jnp.int32).reshape(56 * WP)


def _pack_pair(a_hi, a_lo):
    u = (_round_bf16_bits(a_hi) << 16) | _round_bf16_bits(a_lo)
    return lax.bitcast_convert_type(u, jnp.int32)


def _iw_body(offh_ref, offw_ref, m_ref, ph_ref, pw_ref, o_ref):
    r = pl.program_id(0)
    k = r % K
    ti = (k // 3).astype(jnp.float32)
    tj = (k % 3).astype(jnp.float32)
    h = ph_ref[0, :] + ti + offh_ref[0, 0, 0, :]
    w = pw_ref[0, :] + tj + offw_ref[0, 0, 0, :]
    m = m_ref[0, 0, 0, :]

    h0f = jnp.floor(h)
    w0f = jnp.floor(w)
    lh = h - h0f
    lw = w - w0f
    h1f = h0f + 1.0
    w1f = w0f + 1.0
    fmax_h = jnp.float32(H - 1)
    fmax_w = jnp.float32(W - 1)
    vh0 = (h0f >= 0.0) & (h0f <= fmax_h)
    vh1 = (h1f >= 0.0) & (h1f <= fmax_h)
    vw0 = (w0f >= 0.0) & (w0f <= fmax_w)
    vw1 = (w1f >= 0.0) & (w1f <= fmax_w)
    h0c = jnp.clip(h0f, 0.0, fmax_h).astype(jnp.int32)
    h1c = jnp.clip(h1f, 0.0, fmax_h).astype(jnp.int32)
    w0c = jnp.clip(w0f, 0.0, fmax_w).astype(jnp.int32)
    w1c = jnp.clip(w1f, 0.0, fmax_w).astype(jnp.int32)

    r0 = h0c * WP
    r1 = h1c * WP
    i00 = r0 + w0c
    i01 = r0 + w1c
    i10 = r1 + w0c
    i11 = r1 + w1c
    zero = jnp.float32(0.0)
    a0 = jnp.where(vh0, (1.0 - lh) * m, zero)
    a1 = jnp.where(vh1, lh * m, zero)
    b0 = jnp.where(vw0, 1.0 - lw, zero)
    b1 = jnp.where(vw1, lw, zero)

    o_ref[0, 0, 0, :] = i00 | (i01 << 16)
    o_ref[0, 0, 1, :] = i10 | (i11 << 16)
    o_ref[0, 0, 2, :] = _pack_pair(a0, a1)
    o_ref[0, 0, 3, :] = _pack_pair(b0, b1)


def _make_tc_kernels(interpret=False):
    pack = pl.pallas_call(
        _pack_body,
        grid=(B * CP, 4),
        in_specs=[pl.BlockSpec((1, 1, 2, 56, W),
                               lambda r, c: (r // CP, r % CP, 0, c, 0))],
        out_specs=pl.BlockSpec((1, 1, 56 * WP), lambda r, c: (r, 0, c)),
        out_shape=jax.ShapeDtypeStruct((B * CP, 1, H * WP), jnp.int32),
        interpret=interpret,
    )
    iw = pl.pallas_call(
        _iw_body,
        grid=(B * K, HW // CHUNK),
        in_specs=[
            pl.BlockSpec((1, 1, 1, CHUNK), lambda r, c: (r // K, 2 * (r % K), 0, c)),
            pl.BlockSpec((1, 1, 1, CHUNK), lambda r, c: (r // K, 2 * (r % K) + 1, 0, c)),
            pl.BlockSpec((1, 1, 1, CHUNK), lambda r, c: (r // K, r % K, 0, c)),
            pl.BlockSpec((1, CHUNK), lambda r, c: (0, c)),
            pl.BlockSpec((1, CHUNK), lambda r, c: (0, c)),
        ],
        out_specs=pl.BlockSpec((1, 1, 4, CHUNK), lambda r, c: (r // K, r % K, 0, c)),
        out_shape=jax.ShapeDtypeStruct((B, K, 4, HW), jnp.int32),
        interpret=interpret,
    )
    return pack, iw


_LOWMASK = np.int32(0xFFFF)


def _bf16_hi(word):
    return plsc.bitcast(word, jnp.float32)


def _bf16_lo(word):
    return plsc.bitcast(word << 16, jnp.float32)


def _sc_task(planes, iw, out, iw_v, plane_v, out_v, sems, task):
    b = task // NBLK
    blk = task - b * NBLK
    base = pl.multiple_of(blk * P, 128)
    row0 = pl.multiple_of(blk * 8, 8)
    pltpu.sync_copy(iw.at[b, :, :, pl.ds(base, P)], iw_v)

    def out_copy(slot, ck0, start):
        for ch in range(2):
            d = pltpu.make_async_copy(
                out_v.at[slot, ch],
                out.at[ck0 + ch * K, b, pl.ds(row0, 8), :],
                sems[slot])
            if start:
                d.start()
            else:
                d.wait()

    def cp_body(cp, carry):
        pltpu.sync_copy(planes.at[b * CP + cp, 0], plane_v)
        for k in range(K):
            slot = k % 2
            ck0 = (2 * cp) * K + k

            @pl.when((cp > 0) | (k >= 2))
            def _():
                out_copy(slot, ck0, start=False)

            @plsc.parallel_loop(0, NV, unroll=4)
            def v_body(v):
                s = pl.ds(v * 16, 16)
                r = v // 14
                cs = pl.ds((v - r * 14) * 16, 16)
                idxa = iw_v[k, 0, s]
                idxb = iw_v[k, 1, s]
                wa = iw_v[k, 2, s]
                wb = iw_v[k, 3, s]
                i00 = idxa & _LOWMASK
                i01 = lax.shift_right_logical(idxa, 16)
                i10 = idxb & _LOWMASK
                i11 = lax.shift_right_logical(idxb, 16)
                a0 = _bf16_hi(wa)
                a1 = _bf16_lo(wa)
                b0 = _bf16_hi(wb)
                b1 = _bf16_lo(wb)
                v00 = plsc.load_gather(plane_v, [i00])
                v01 = plsc.load_gather(plane_v, [i01])
                v10 = plsc.load_gather(plane_v, [i10])
                v11 = plsc.load_gather(plane_v, [i11])
                t0 = (_bf16_hi(v00) * b0 + _bf16_hi(v01) * b1) * a0 \
                    + (_bf16_hi(v10) * b0 + _bf16_hi(v11) * b1) * a1
                t1 = (_bf16_lo(v00) * b0 + _bf16_lo(v01) * b1) * a0 \
                    + (_bf16_lo(v10) * b0 + _bf16_lo(v11) * b1) * a1
                out_v[slot, 0, r, cs] = t0
                out_v[slot, 1, r, cs] = t1
            out_copy(slot, ck0, start=True)
        return carry
    lax.fori_loop(0, CP, cp_body, 0)

    last_ck0 = (2 * (CP - 1)) * K
    for k in range(K - 2, K):
        out_copy(k % 2, last_ck0 + k, start=False)


def _sc_body(planes, iw, out, iw_v, plane_v, out_v, s0, s1):
    sems = (s0, s1)
    wid = lax.axis_index("s") * 2 + lax.axis_index("c")
    _sc_task(planes, iw, out, iw_v, plane_v, out_v, sems, wid)

    @pl.when(wid < NTASK - 32)
    def _():
        _sc_task(planes, iw, out, iw_v, plane_v, out_v, sems, wid + 32)


def _make_sc_kernel(interpret=False):
    mesh = plsc.VectorSubcoreMesh(core_axis_name="c", subcore_axis_name="s")
    return functools.partial(
        pl.kernel,
        out_type=jax.ShapeDtypeStruct((C * K, B, H, W), jnp.float32),
        mesh=mesh,
        scratch_types=[
            pltpu.VMEM((K, 4, P), jnp.int32),
            pltpu.VMEM((H * WP,), jnp.int32),
            pltpu.VMEM((2, 2, 8, W), jnp.float32),
            pltpu.SemaphoreType.DMA,
            pltpu.SemaphoreType.DMA,
        ],
        compiler_params=pltpu.CompilerParams(needs_layout_passes=False),
        interpret=interpret,
    )(_sc_body)


def kernel(data_im, offset, mask):
    x5 = data_im.reshape(B, CP, 2, H, W)
    off2 = offset.reshape(B, 2 * K, 1, HW)
    m2 = mask.reshape(B, K, 1, HW)
    pos = jnp.arange(HW, dtype=jnp.int32)
    ph = ((pos // W) - 1).astype(jnp.float32).reshape(1, HW)
    pw = ((pos % W) - 1).astype(jnp.float32).reshape(1, HW)

    pack, iw_fn = _make_tc_kernels()
    planes = pack(x5)
    iw = iw_fn(off2, off2, m2, ph, pw)
    return _make_sc_kernel()(planes, iw)


# speedup vs baseline: 1.0839x; 1.0839x over previous
import functools

import numpy as np

import jax
import jax.numpy as jnp
from jax import lax
from jax.experimental import pallas as pl
from jax.experimental.pallas import tpu as pltpu
from jax.experimental.pallas import tpu_sc as plsc

B = 2
C = 96
CP = C // 2
H = W = 224
HW = H * W
K = 9
P = 1792
NV = P // 16
NBLK = HW // P
NTASK = B * NBLK
CHUNK = 12544
WP = 256


def _round_bf16_bits(x):
    u = lax.bitcast_convert_type(x, jnp.uint32)
    return (u + jnp.uint32(0x8000)) >> 16


def _pack_body(x_ref, o_ref):
    x0 = x_ref[0, 0, 0]
    x1 = x_ref[0, 0, 1]
    word = (_round_bf16_bits(x0) << 16) | _round_bf16_bits(x1)
    wp = jnp.concatenate([word, jnp.zeros((H, WP - W), jnp.uint32)], axis=1)
    o_ref[0, 0, :] = lax.bitcast_convert_type(wp, jnp.int32).reshape(H * WP)


def _pack_pair(a_hi, a_lo):
    u = (_round_bf16_bits(a_hi) << 16) | _round_bf16_bits(a_lo)
    return lax.bitcast_convert_type(u, jnp.int32)


def _iw_body(offh_ref, offw_ref, m_ref, ph_ref, pw_ref, o_ref):
    r = pl.program_id(0)
    k = r % K
    ti = (k // 3).astype(jnp.float32)
    tj = (k % 3).astype(jnp.float32)
    h = ph_ref[0, :] + ti + offh_ref[0, 0, 0, :]
    w = pw_ref[0, :] + tj + offw_ref[0, 0, 0, :]
    m = m_ref[0, 0, 0, :]

    h0f = jnp.floor(h)
    w0f = jnp.floor(w)
    lh = h - h0f
    lw = w - w0f
    h1f = h0f + 1.0
    w1f = w0f + 1.0
    fmax_h = jnp.float32(H - 1)
    fmax_w = jnp.float32(W - 1)
    vh0 = (h0f >= 0.0) & (h0f <= fmax_h)
    vh1 = (h1f >= 0.0) & (h1f <= fmax_h)
    vw0 = (w0f >= 0.0) & (w0f <= fmax_w)
    vw1 = (w1f >= 0.0) & (w1f <= fmax_w)
    h0c = jnp.clip(h0f, 0.0, fmax_h).astype(jnp.int32)
    h1c = jnp.clip(h1f, 0.0, fmax_h).astype(jnp.int32)
    w0c = jnp.clip(w0f, 0.0, fmax_w).astype(jnp.int32)
    w1c = jnp.clip(w1f, 0.0, fmax_w).astype(jnp.int32)

    r0 = h0c * WP
    r1 = h1c * WP
    i00 = r0 + w0c
    i01 = r0 + w1c
    i10 = r1 + w0c
    i11 = r1 + w1c
    zero = jnp.float32(0.0)
    a0 = jnp.where(vh0, (1.0 - lh) * m, zero)
    a1 = jnp.where(vh1, lh * m, zero)
    b0 = jnp.where(vw0, 1.0 - lw, zero)
    b1 = jnp.where(vw1, lw, zero)

    o_ref[0, 0, 0, :] = i00 | (i01 << 16)
    o_ref[0, 0, 1, :] = i10 | (i11 << 16)
    o_ref[0, 0, 2, :] = _pack_pair(a0, a1)
    o_ref[0, 0, 3, :] = _pack_pair(b0, b1)


def _make_tc_kernels(interpret=False):
    pack = pl.pallas_call(
        _pack_body,
        grid=(B * CP,),
        in_specs=[pl.BlockSpec((1, 1, 2, H, W),
                               lambda r: (r // CP, r % CP, 0, 0, 0))],
        out_specs=pl.BlockSpec((1, 1, H * WP), lambda r: (r, 0, 0)),
        out_shape=jax.ShapeDtypeStruct((B * CP, 1, H * WP), jnp.int32),
        interpret=interpret,
    )
    iw = pl.pallas_call(
        _iw_body,
        grid=(B * K, HW // CHUNK),
        in_specs=[
            pl.BlockSpec((1, 1, 1, CHUNK), lambda r, c: (r // K, 2 * (r % K), 0, c)),
            pl.BlockSpec((1, 1, 1, CHUNK), lambda r, c: (r // K, 2 * (r % K) + 1, 0, c)),
            pl.BlockSpec((1, 1, 1, CHUNK), lambda r, c: (r // K, r % K, 0, c)),
            pl.BlockSpec((1, CHUNK), lambda r, c: (0, c)),
            pl.BlockSpec((1, CHUNK), lambda r, c: (0, c)),
        ],
        out_specs=pl.BlockSpec((1, 1, 4, CHUNK), lambda r, c: (r // K, r % K, 0, c)),
        out_shape=jax.ShapeDtypeStruct((B, K, 4, HW), jnp.int32),
        interpret=interpret,
    )
    return pack, iw


_LOWMASK = np.int32(0xFFFF)


def _bf16_hi(word):
    return plsc.bitcast(word, jnp.float32)


def _bf16_lo(word):
    return plsc.bitcast(word << 16, jnp.float32)


def _sc_task(planes, iw, out, iw_v, plane_v, out_v, sems, task):
    b = task // NBLK
    blk = task - b * NBLK
    base = pl.multiple_of(blk * P, 128)
    row0 = pl.multiple_of(blk * 8, 8)
    pltpu.sync_copy(iw.at[b, :, :, pl.ds(base, P)], iw_v)

    def out_copy(slot, ck0, start):
        for ch in range(2):
            d = pltpu.make_async_copy(
                out_v.at[slot, ch],
                out.at[ck0 + ch * K, b, pl.ds(row0, 8), :],
                sems[slot])
            if start:
                d.start()
            else:
                d.wait()

    def cp_body(cp, carry):
        pltpu.sync_copy(planes.at[b * CP + cp, 0], plane_v)
        for k in range(K):
            slot = k % 2
            ck0 = (2 * cp) * K + k

            @pl.when((cp > 0) | (k >= 2))
            def _():
                out_copy(slot, ck0, start=False)

            @plsc.parallel_loop(0, NV, unroll=8)
            def v_body(v):
                s = pl.ds(v * 16, 16)
                r = v // 14
                cs = pl.ds((v - r * 14) * 16, 16)
                idxa = iw_v[k, 0, s]
                idxb = iw_v[k, 1, s]
                wa = iw_v[k, 2, s]
                wb = iw_v[k, 3, s]
                i00 = idxa & _LOWMASK
                i01 = lax.shift_right_logical(idxa, 16)
                i10 = idxb & _LOWMASK
                i11 = lax.shift_right_logical(idxb, 16)
                a0 = _bf16_hi(wa)
                a1 = _bf16_lo(wa)
                b0 = _bf16_hi(wb)
                b1 = _bf16_lo(wb)
                v00 = plsc.load_gather(plane_v, [i00])
                v01 = plsc.load_gather(plane_v, [i01])
                v10 = plsc.load_gather(plane_v, [i10])
                v11 = plsc.load_gather(plane_v, [i11])
                t0 = (_bf16_hi(v00) * b0 + _bf16_hi(v01) * b1) * a0 \
                    + (_bf16_hi(v10) * b0 + _bf16_hi(v11) * b1) * a1
                t1 = (_bf16_lo(v00) * b0 + _bf16_lo(v01) * b1) * a0 \
                    + (_bf16_lo(v10) * b0 + _bf16_lo(v11) * b1) * a1
                out_v[slot, 0, r, cs] = t0
                out_v[slot, 1, r, cs] = t1
            out_copy(slot, ck0, start=True)
        return carry
    lax.fori_loop(0, CP, cp_body, 0)

    last_ck0 = (2 * (CP - 1)) * K
    for k in range(K - 2, K):
        out_copy(k % 2, last_ck0 + k, start=False)


def _sc_body(planes, iw, out, iw_v, plane_v, out_v, s0, s1):
    sems = (s0, s1)
    wid = lax.axis_index("s") * 2 + lax.axis_index("c")
    _sc_task(planes, iw, out, iw_v, plane_v, out_v, sems, wid)

    @pl.when(wid < NTASK - 32)
    def _():
        _sc_task(planes, iw, out, iw_v, plane_v, out_v, sems, wid + 32)


def _make_sc_kernel(interpret=False):
    mesh = plsc.VectorSubcoreMesh(core_axis_name="c", subcore_axis_name="s")
    return functools.partial(
        pl.kernel,
        out_type=jax.ShapeDtypeStruct((C * K, B, H, W), jnp.float32),
        mesh=mesh,
        scratch_types=[
            pltpu.VMEM((K, 4, P), jnp.int32),
            pltpu.VMEM((H * WP,), jnp.int32),
            pltpu.VMEM((2, 2, 8, W), jnp.float32),
            pltpu.SemaphoreType.DMA,
            pltpu.SemaphoreType.DMA,
        ],
        compiler_params=pltpu.CompilerParams(needs_layout_passes=False),
        interpret=interpret,
    )(_sc_body)


def kernel(data_im, offset, mask):
    x5 = data_im.reshape(B, CP, 2, H, W)
    off2 = offset.reshape(B, 2 * K, 1, HW)
    m2 = mask.reshape(B, K, 1, HW)
    pos = jnp.arange(HW, dtype=jnp.int32)
    ph = ((pos // W) - 1).astype(jnp.float32).reshape(1, HW)
    pw = ((pos % W) - 1).astype(jnp.float32).reshape(1, HW)

    pack, iw_fn = _make_tc_kernels()
    planes = pack(x5)
    iw = iw_fn(off2, off2, m2, ph, pw)
    return _make_sc_kernel()(planes, iw)


# speedup vs baseline: 1.1701x; 1.0795x over previous
import functools

import numpy as np

import jax
import jax.numpy as jnp
from jax import lax
from jax.experimental import pallas as pl
from jax.experimental.pallas import tpu as pltpu
from jax.experimental.pallas import tpu_sc as plsc

B = 2
C = 96
CP = C // 2
H = W = 224
HW = H * W
K = 9
P = 1792
NV = P // 16
NBLK = HW // P
NTASK = B * NBLK
NQ = 4
NQTASK = NTASK * NQ
CHUNK = 12544
WP = 256


def _round_bf16_bits(x):
    u = lax.bitcast_convert_type(x, jnp.uint32)
    return (u + jnp.uint32(0x8000)) >> 16


def _pack_body(x_ref, o_ref):
    x0 = x_ref[0, 0, 0]
    x1 = x_ref[0, 0, 1]
    word = (_round_bf16_bits(x0) << 16) | _round_bf16_bits(x1)
    wp = jnp.concatenate([word, jnp.zeros((H, WP - W), jnp.uint32)], axis=1)
    o_ref[0, 0, :] = lax.bitcast_convert_type(wp, jnp.int32).reshape(H * WP)


def _pack_pair(a_hi, a_lo):
    u = (_round_bf16_bits(a_hi) << 16) | _round_bf16_bits(a_lo)
    return lax.bitcast_convert_type(u, jnp.int32)


def _iw_body(offh_ref, offw_ref, m_ref, ph_ref, pw_ref, o_ref):
    r = pl.program_id(0)
    k = r % K
    ti = (k // 3).astype(jnp.float32)
    tj = (k % 3).astype(jnp.float32)
    h = ph_ref[0, :] + ti + offh_ref[0, 0, 0, :]
    w = pw_ref[0, :] + tj + offw_ref[0, 0, 0, :]
    m = m_ref[0, 0, 0, :]

    h0f = jnp.floor(h)
    w0f = jnp.floor(w)
    lh = h - h0f
    lw = w - w0f
    h1f = h0f + 1.0
    w1f = w0f + 1.0
    fmax_h = jnp.float32(H - 1)
    fmax_w = jnp.float32(W - 1)
    vh0 = (h0f >= 0.0) & (h0f <= fmax_h)
    vh1 = (h1f >= 0.0) & (h1f <= fmax_h)
    vw0 = (w0f >= 0.0) & (w0f <= fmax_w)
    vw1 = (w1f >= 0.0) & (w1f <= fmax_w)
    h0c = jnp.clip(h0f, 0.0, fmax_h).astype(jnp.int32)
    h1c = jnp.clip(h1f, 0.0, fmax_h).astype(jnp.int32)
    w0c = jnp.clip(w0f, 0.0, fmax_w).astype(jnp.int32)
    w1c = jnp.clip(w1f, 0.0, fmax_w).astype(jnp.int32)

    r0 = h0c * WP
    r1 = h1c * WP
    i00 = r0 + w0c
    i01 = r0 + w1c
    i10 = r1 + w0c
    i11 = r1 + w1c
    zero = jnp.float32(0.0)
    a0 = jnp.where(vh0, (1.0 - lh) * m, zero)
    a1 = jnp.where(vh1, lh * m, zero)
    b0 = jnp.where(vw0, 1.0 - lw, zero)
    b1 = jnp.where(vw1, lw, zero)

    o_ref[0, 0, 0, :] = i00 | (i01 << 16)
    o_ref[0, 0, 1, :] = i10 | (i11 << 16)
    o_ref[0, 0, 2, :] = _pack_pair(a0, a1)
    o_ref[0, 0, 3, :] = _pack_pair(b0, b1)


def _make_tc_kernels(interpret=False):
    pack = pl.pallas_call(
        _pack_body,
        grid=(B * CP,),
        in_specs=[pl.BlockSpec((1, 1, 2, H, W),
                               lambda r: (r // CP, r % CP, 0, 0, 0))],
        out_specs=pl.BlockSpec((1, 1, H * WP), lambda r: (r, 0, 0)),
        out_shape=jax.ShapeDtypeStruct((B * CP, 1, H * WP), jnp.int32),
        interpret=interpret,
    )
    iw = pl.pallas_call(
        _iw_body,
        grid=(B * K, HW // CHUNK),
        in_specs=[
            pl.BlockSpec((1, 1, 1, CHUNK), lambda r, c: (r // K, 2 * (r % K), 0, c)),
            pl.BlockSpec((1, 1, 1, CHUNK), lambda r, c: (r // K, 2 * (r % K) + 1, 0, c)),
            pl.BlockSpec((1, 1, 1, CHUNK), lambda r, c: (r // K, r % K, 0, c)),
            pl.BlockSpec((1, CHUNK), lambda r, c: (0, c)),
            pl.BlockSpec((1, CHUNK), lambda r, c: (0, c)),
        ],
        out_specs=pl.BlockSpec((1, 1, 4, CHUNK), lambda r, c: (r // K, r % K, 0, c)),
        out_shape=jax.ShapeDtypeStruct((B, K, 4, HW), jnp.int32),
        interpret=interpret,
    )
    return pack, iw


_LOWMASK = np.int32(0xFFFF)


def _bf16_hi(word):
    return plsc.bitcast(word, jnp.float32)


def _bf16_lo(word):
    return plsc.bitcast(word << 16, jnp.float32)


def _sc_task(planes, iw, out, iw_v, plane_v, out_v, sems, task):
    bb = task // NQ
    q = task - bb * NQ
    b = bb // NBLK
    blk = bb - b * NBLK
    cp0 = q * (CP // NQ)
    base = pl.multiple_of(blk * P, 128)
    row0 = pl.multiple_of(blk * 8, 8)
    pltpu.sync_copy(iw.at[b, :, :, pl.ds(base, P)], iw_v)

    def out_copy(slot, ck0, start):
        for ch in range(2):
            d = pltpu.make_async_copy(
                out_v.at[slot, ch],
                out.at[ck0 + ch * K, b, pl.ds(row0, 8), :],
                sems[slot])
            if start:
                d.start()
            else:
                d.wait()

    def cp_body(cp, carry):
        pltpu.sync_copy(planes.at[b * CP + cp0 + cp, 0], plane_v)
        for k in range(K):
            slot = k % 2
            ck0 = (2 * (cp0 + cp)) * K + k

            @pl.when((cp > 0) | (k >= 2))
            def _():
                out_copy(slot, ck0, start=False)

            @plsc.parallel_loop(0, NV, unroll=8)
            def v_body(v):
                s = pl.ds(v * 16, 16)
                r = v // 14
                cs = pl.ds((v - r * 14) * 16, 16)
                idxa = iw_v[k, 0, s]
                idxb = iw_v[k, 1, s]
                wa = iw_v[k, 2, s]
                wb = iw_v[k, 3, s]
                i00 = idxa & _LOWMASK
                i01 = lax.shift_right_logical(idxa, 16)
                i10 = idxb & _LOWMASK
                i11 = lax.shift_right_logical(idxb, 16)
                a0 = _bf16_hi(wa)
                a1 = _bf16_lo(wa)
                b0 = _bf16_hi(wb)
                b1 = _bf16_lo(wb)
                v00 = plsc.load_gather(plane_v, [i00])
                v01 = plsc.load_gather(plane_v, [i01])
                v10 = plsc.load_gather(plane_v, [i10])
                v11 = plsc.load_gather(plane_v, [i11])
                t0 = (_bf16_hi(v00) * b0 + _bf16_hi(v01) * b1) * a0 \
                    + (_bf16_hi(v10) * b0 + _bf16_hi(v11) * b1) * a1
                t1 = (_bf16_lo(v00) * b0 + _bf16_lo(v01) * b1) * a0 \
                    + (_bf16_lo(v10) * b0 + _bf16_lo(v11) * b1) * a1
                out_v[slot, 0, r, cs] = t0
                out_v[slot, 1, r, cs] = t1
            out_copy(slot, ck0, start=True)
        return carry
    lax.fori_loop(0, CP // NQ, cp_body, 0)

    last_ck0 = (2 * (cp0 + CP // NQ - 1)) * K
    for k in range(K - 2, K):
        out_copy(k % 2, last_ck0 + k, start=False)


def _sc_body(planes, iw, out, iw_v, plane_v, out_v, s0, s1):
    sems = (s0, s1)
    wid = lax.axis_index("s") * 2 + lax.axis_index("c")
    for j in range(NQTASK // 32):
        _sc_task(planes, iw, out, iw_v, plane_v, out_v, sems, wid + 32 * j)


def _make_sc_kernel(interpret=False):
    mesh = plsc.VectorSubcoreMesh(core_axis_name="c", subcore_axis_name="s")
    return functools.partial(
        pl.kernel,
        out_type=jax.ShapeDtypeStruct((C * K, B, H, W), jnp.float32),
        mesh=mesh,
        scratch_types=[
            pltpu.VMEM((K, 4, P), jnp.int32),
            pltpu.VMEM((H * WP,), jnp.int32),
            pltpu.VMEM((2, 2, 8, W), jnp.float32),
            pltpu.SemaphoreType.DMA,
            pltpu.SemaphoreType.DMA,
        ],
        compiler_params=pltpu.CompilerParams(needs_layout_passes=False),
        interpret=interpret,
    )(_sc_body)


def kernel(data_im, offset, mask):
    x5 = data_im.reshape(B, CP, 2, H, W)
    off2 = offset.reshape(B, 2 * K, 1, HW)
    m2 = mask.reshape(B, K, 1, HW)
    pos = jnp.arange(HW, dtype=jnp.int32)
    ph = ((pos // W) - 1).astype(jnp.float32).reshape(1, HW)
    pw = ((pos % W) - 1).astype(jnp.float32).reshape(1, HW)

    pack, iw_fn = _make_tc_kernels()
    planes = pack(x5)
    iw = iw_fn(off2, off2, m2, ph, pw)
    return _make_sc_kernel()(planes, iw)


# speedup vs baseline: 1.2113x; 1.0352x over previous
import functools

import numpy as np

import jax
import jax.numpy as jnp
from jax import lax
from jax.experimental import pallas as pl
from jax.experimental.pallas import tpu as pltpu
from jax.experimental.pallas import tpu_sc as plsc

B = 2
C = 96
CP = C // 2
H = W = 224
HW = H * W
K = 9
P = 1792
NV = P // 16
NBLK = HW // P
NTASK = B * NBLK
NQ = 4
NQTASK = NTASK * NQ
CHUNK = 12544
WP = 256


def _round_bf16_bits(x):
    u = lax.bitcast_convert_type(x, jnp.uint32)
    return (u + jnp.uint32(0x8000)) >> 16


def _pack_body(x_ref, o_ref):
    x0 = x_ref[0, 0, 0]
    x1 = x_ref[0, 0, 1]
    word = (_round_bf16_bits(x0) << 16) | _round_bf16_bits(x1)
    wp = jnp.concatenate([word, jnp.zeros((H, WP - W), jnp.uint32)], axis=1)
    o_ref[0, 0, :] = lax.bitcast_convert_type(wp, jnp.int32).reshape(H * WP)


def _pack_pair(a_hi, a_lo):
    u = (_round_bf16_bits(a_hi) << 16) | _round_bf16_bits(a_lo)
    return lax.bitcast_convert_type(u, jnp.int32)


def _iw_body(offh_ref, offw_ref, m_ref, ph_ref, pw_ref, o_ref):
    r = pl.program_id(0)
    k = r % K
    ti = (k // 3).astype(jnp.float32)
    tj = (k % 3).astype(jnp.float32)
    h = ph_ref[:, :] + ti + offh_ref[0, 0]
    w = pw_ref[:, :] + tj + offw_ref[0, 0]
    m = m_ref[0, 0]

    h0f = jnp.floor(h)
    w0f = jnp.floor(w)
    lh = h - h0f
    lw = w - w0f
    h1f = h0f + 1.0
    w1f = w0f + 1.0
    fmax_h = jnp.float32(H - 1)
    fmax_w = jnp.float32(W - 1)
    vh0 = (h0f >= 0.0) & (h0f <= fmax_h)
    vh1 = (h1f >= 0.0) & (h1f <= fmax_h)
    vw0 = (w0f >= 0.0) & (w0f <= fmax_w)
    vw1 = (w1f >= 0.0) & (w1f <= fmax_w)
    h0c = jnp.clip(h0f, 0.0, fmax_h).astype(jnp.int32)
    h1c = jnp.clip(h1f, 0.0, fmax_h).astype(jnp.int32)
    w0c = jnp.clip(w0f, 0.0, fmax_w).astype(jnp.int32)
    w1c = jnp.clip(w1f, 0.0, fmax_w).astype(jnp.int32)

    r0 = h0c * WP
    r1 = h1c * WP
    i00 = r0 + w0c
    i01 = r0 + w1c
    i10 = r1 + w0c
    i11 = r1 + w1c
    zero = jnp.float32(0.0)
    a0 = jnp.where(vh0, (1.0 - lh) * m, zero)
    a1 = jnp.where(vh1, lh * m, zero)
    b0 = jnp.where(vw0, 1.0 - lw, zero)
    b1 = jnp.where(vw1, lw, zero)

    o_ref[0, 0, 0] = i00 | (i01 << 16)
    o_ref[0, 0, 1] = i10 | (i11 << 16)
    o_ref[0, 0, 2] = _pack_pair(a0, a1)
    o_ref[0, 0, 3] = _pack_pair(b0, b1)


def _make_tc_kernels(interpret=False):
    pack = pl.pallas_call(
        _pack_body,
        grid=(B * CP,),
        in_specs=[pl.BlockSpec((1, 1, 2, H, W),
                               lambda r: (r // CP, r % CP, 0, 0, 0))],
        out_specs=pl.BlockSpec((1, 1, H * WP), lambda r: (r, 0, 0)),
        out_shape=jax.ShapeDtypeStruct((B * CP, 1, H * WP), jnp.int32),
        interpret=interpret,
    )
    iw = pl.pallas_call(
        _iw_body,
        grid=(B * K,),
        in_specs=[
            pl.BlockSpec((1, 1, 8, HW // 8), lambda r: (r // K, 2 * (r % K), 0, 0)),
            pl.BlockSpec((1, 1, 8, HW // 8), lambda r: (r // K, 2 * (r % K) + 1, 0, 0)),
            pl.BlockSpec((1, 1, 8, HW // 8), lambda r: (r // K, r % K, 0, 0)),
            pl.BlockSpec((8, HW // 8), lambda r: (0, 0)),
            pl.BlockSpec((8, HW // 8), lambda r: (0, 0)),
        ],
        out_specs=pl.BlockSpec((1, 1, 4, 8, HW // 8), lambda r: (r // K, r % K, 0, 0, 0)),
        out_shape=jax.ShapeDtypeStruct((B, K, 4, 8, HW // 8), jnp.int32),
        interpret=interpret,
    )
    return pack, iw


_LOWMASK = np.int32(0xFFFF)


def _bf16_hi(word):
    return plsc.bitcast(word, jnp.float32)


def _bf16_lo(word):
    return plsc.bitcast(word << 16, jnp.float32)


def _sc_task(planes, iw, out, iw_v, plane_v, out_v, sems, task):
    bb = task // NQ
    q = task - bb * NQ
    b = bb // NBLK
    blk = bb - b * NBLK
    cp0 = q * (CP // NQ)
    base = pl.multiple_of(blk * P, 128)
    row0 = pl.multiple_of(blk * 8, 8)
    pltpu.sync_copy(iw.at[b, :, :, pl.ds(base, P)], iw_v)

    def out_copy(slot, ck0, start):
        for ch in range(2):
            d = pltpu.make_async_copy(
                out_v.at[slot, ch],
                out.at[ck0 + ch * K, b, pl.ds(row0, 8), :],
                sems[slot])
            if start:
                d.start()
            else:
                d.wait()

    def cp_body(cp, carry):
        pltpu.sync_copy(planes.at[b * CP + cp0 + cp, 0], plane_v)
        for k in range(K):
            slot = k % 2
            ck0 = (2 * (cp0 + cp)) * K + k

            @pl.when((cp > 0) | (k >= 2))
            def _():
                out_copy(slot, ck0, start=False)

            @plsc.parallel_loop(0, NV, unroll=8)
            def v_body(v):
                s = pl.ds(v * 16, 16)
                r = v // 14
                cs = pl.ds((v - r * 14) * 16, 16)
                idxa = iw_v[k, 0, s]
                idxb = iw_v[k, 1, s]
                wa = iw_v[k, 2, s]
                wb = iw_v[k, 3, s]
                i00 = idxa & _LOWMASK
                i01 = lax.shift_right_logical(idxa, 16)
                i10 = idxb & _LOWMASK
                i11 = lax.shift_right_logical(idxb, 16)
                a0 = _bf16_hi(wa)
                a1 = _bf16_lo(wa)
                b0 = _bf16_hi(wb)
                b1 = _bf16_lo(wb)
                v00 = plsc.load_gather(plane_v, [i00])
                v01 = plsc.load_gather(plane_v, [i01])
                v10 = plsc.load_gather(plane_v, [i10])
                v11 = plsc.load_gather(plane_v, [i11])
                t0 = (_bf16_hi(v00) * b0 + _bf16_hi(v01) * b1) * a0 \
                    + (_bf16_hi(v10) * b0 + _bf16_hi(v11) * b1) * a1
                t1 = (_bf16_lo(v00) * b0 + _bf16_lo(v01) * b1) * a0 \
                    + (_bf16_lo(v10) * b0 + _bf16_lo(v11) * b1) * a1
                out_v[slot, 0, r, cs] = t0
                out_v[slot, 1, r, cs] = t1
            out_copy(slot, ck0, start=True)
        return carry
    lax.fori_loop(0, CP // NQ, cp_body, 0)

    last_ck0 = (2 * (cp0 + CP // NQ - 1)) * K
    for k in range(K - 2, K):
        out_copy(k % 2, last_ck0 + k, start=False)


def _sc_body(planes, iw, out, iw_v, plane_v, out_v, s0, s1):
    sems = (s0, s1)
    wid = lax.axis_index("s") * 2 + lax.axis_index("c")
    for j in range(NQTASK // 32):
        _sc_task(planes, iw, out, iw_v, plane_v, out_v, sems, wid + 32 * j)


def _make_sc_kernel(interpret=False):
    mesh = plsc.VectorSubcoreMesh(core_axis_name="c", subcore_axis_name="s")
    return functools.partial(
        pl.kernel,
        out_type=jax.ShapeDtypeStruct((C * K, B, H, W), jnp.float32),
        mesh=mesh,
        scratch_types=[
            pltpu.VMEM((K, 4, P), jnp.int32),
            pltpu.VMEM((H * WP,), jnp.int32),
            pltpu.VMEM((2, 2, 8, W), jnp.float32),
            pltpu.SemaphoreType.DMA,
            pltpu.SemaphoreType.DMA,
        ],
        compiler_params=pltpu.CompilerParams(needs_layout_passes=False),
        interpret=interpret,
    )(_sc_body)


def kernel(data_im, offset, mask):
    x5 = data_im.reshape(B, CP, 2, H, W)
    off2 = offset.reshape(B, 2 * K, 8, HW // 8)
    m2 = mask.reshape(B, K, 8, HW // 8)
    pos = jnp.arange(HW, dtype=jnp.int32)
    ph = ((pos // W) - 1).astype(jnp.float32).reshape(8, HW // 8)
    pw = ((pos % W) - 1).astype(jnp.float32).reshape(8, HW // 8)

    pack, iw_fn = _make_tc_kernels()
    planes = pack(x5)
    iw = iw_fn(off2, off2, m2, ph, pw).reshape(B, K, 4, HW)
    return _make_sc_kernel()(planes, iw)
